# baseline (speedup 1.0000x reference)
import functools

import jax
import jax.numpy as jnp
from jax import lax
from jax.experimental import pallas as pl
from jax.experimental.pallas import tpu as pltpu
from jax.experimental.pallas import tpu_sc as plsc

B = 16384
D = 128
L = 16
NC = 2
NS = 16
NW = NC * NS
BPW = B // NW
CHUNK = 64
ROW_UNROLL = 8
PCOLS = 128
PROWS = B * L // PCOLS
PROWS_PW = BPW * L // PCOLS


def _sc_gather_dot(emb, ctx, src_idx, dst_idx):
    mesh = plsc.VectorSubcoreMesh(core_axis_name="c", subcore_axis_name="s")
    out_t = jax.ShapeDtypeStruct((PROWS, PCOLS), jnp.float32)

    @functools.partial(
        pl.kernel,
        out_type=out_t,
        mesh=mesh,
        scratch_types=[
            pltpu.VMEM((BPW,), jnp.int32),
            pltpu.VMEM((BPW,), jnp.int32),
            pltpu.VMEM((CHUNK, D), jnp.float32),
            pltpu.VMEM((CHUNK, D), jnp.float32),
            pltpu.VMEM((CHUNK, D), jnp.float32),
            pltpu.VMEM((CHUNK, D), jnp.float32),
            pltpu.VMEM((CHUNK, D), jnp.float32),
            pltpu.VMEM((CHUNK, D), jnp.float32),
            pltpu.VMEM((PROWS_PW, PCOLS), jnp.float32),
            pltpu.SemaphoreType.DMA,
            pltpu.SemaphoreType.DMA,
            pltpu.SemaphoreType.DMA,
            pltpu.SemaphoreType.DMA,
            pltpu.SemaphoreType.DMA,
            pltpu.SemaphoreType.DMA,
            pltpu.SemaphoreType.DMA,
            pltpu.SemaphoreType.DMA,
        ],
    )
    def k(emb_hbm, ctx_hbm, src_hbm, dst_hbm, out_hbm,
          sidx_v, didx_v, sbuf0, sbuf1, sbuf2, dbuf0, dbuf1, dbuf2, part_v,
          sem_s0, sem_s1, sem_s2, sem_d0, sem_d1, sem_d2, sem_i0, sem_i1):
        wid = lax.axis_index("s") * NC + lax.axis_index("c")
        base = wid * BPW
        ci0 = pltpu.async_copy(src_hbm.at[pl.ds(base, BPW)], sidx_v, sem_i0)
        ci1 = pltpu.async_copy(dst_hbm.at[pl.ds(base, BPW)], didx_v, sem_i1)
        ci0.wait()
        ci1.wait()

        sbufs = (sbuf0, sbuf1, sbuf2)
        dbufs = (dbuf0, dbuf1, dbuf2)
        ssems = (sem_s0, sem_s1, sem_s2)
        dsems = (sem_d0, sem_d1, sem_d2)
        DEPTH = 3
        CHUNKS = (64, 64, 64, 64, 64, 64, 64, 64)
        OFFS = tuple(sum(CHUNKS[:i]) for i in range(len(CHUNKS)))
        NCH = len(CHUNKS)

        def start(c):
            off = OFFS[c]
            n = CHUNKS[c]
            cs = pltpu.async_copy(
                emb_hbm.at[sidx_v.at[pl.ds(off, n)]],
                sbufs[c % DEPTH].at[pl.ds(0, n)],
                ssems[c % DEPTH])
            cd = pltpu.async_copy(
                ctx_hbm.at[didx_v.at[pl.ds(off, n)]],
                dbufs[c % DEPTH].at[pl.ds(0, n)],
                dsems[c % DEPTH])
            return cs, cd

        pend = [start(c) for c in range(DEPTH)]
        wouts = []
        for c in range(NCH):
            cs, cd = pend[c % DEPTH]
            cs.wait()
            cd.wait()
            sb = sbufs[c % DEPTH]
            db = dbufs[c % DEPTH]
            pbase = OFFS[c]

            @plsc.parallel_loop(0, CHUNKS[c], step=ROW_UNROLL, unroll=3)
            def _(r0):
                pline = (pbase + r0) // 8
                for p in range(ROW_UNROLL // 2):
                    ra = r0 + 2 * p
                    rb = ra + 1
                    acca = sb[ra, pl.ds(0, L)] * db[ra, pl.ds(0, L)]
                    accb = sb[rb, pl.ds(0, L)] * db[rb, pl.ds(0, L)]
                    for g in range(1, D // L):
                        acca += sb[ra, pl.ds(g * L, L)] * db[ra, pl.ds(g * L, L)]
                        accb += sb[rb, pl.ds(g * L, L)] * db[rb, pl.ds(g * L, L)]
                    ua, ub = 2 * p, 2 * p + 1
                    part_v[pline + ua // 8, pl.ds((ua % 8) * L, L)] = acca
                    part_v[pline + ub // 8, pl.ds((ub % 8) * L, L)] = accb

            if c + DEPTH < NCH:
                pend[c % DEPTH] = start(c + DEPTH)

            lo = OFFS[c] // 8
            n = CHUNKS[c] // 8
            wouts.append(pltpu.async_copy(
                part_v.at[pl.ds(lo, n)],
                out_hbm.at[pl.ds(wid * PROWS_PW + lo, n)],
                sem_i0))

        for w in wouts:
            w.wait()

    return k(emb, ctx, src_idx, dst_idx)


def _tc_loss_body(p_ref, o_ref):
    y = p_ref[...]
    r_grp = jax.lax.broadcasted_iota(jnp.int32, (PCOLS, PCOLS), 0) // L
    c_grp = jax.lax.broadcasted_iota(jnp.int32, (PCOLS, PCOLS), 1) // L
    g = (r_grp == c_grp).astype(jnp.float32)
    dot = jax.lax.dot_general(y, g, (((1,), (0,)), ((), ())),
                              preferred_element_type=jnp.float32)
    sp = jax.nn.softplus(-dot)
    o_ref[...] = (jnp.sum(sp) * (1.0 / (B * L))).reshape(1, 1)


def _tc_loss(p):
    out = pl.pallas_call(
        _tc_loss_body,
        out_shape=jax.ShapeDtypeStruct((1, 1), jnp.float32),
    )(p)
    return out[0, 0]


def kernel(src_nodes, dst_nodes, embedding, context_embedding):
    part = _sc_gather_dot(
        embedding,
        context_embedding,
        src_nodes.astype(jnp.int32),
        dst_nodes.astype(jnp.int32),
    )
    return _tc_loss(part)


# speedup vs baseline: 1.1359x; 1.1359x over previous
import functools

import jax
import jax.numpy as jnp
from jax import lax
from jax.experimental import pallas as pl
from jax.experimental.pallas import tpu as pltpu
from jax.experimental.pallas import tpu_sc as plsc

B = 16384
D = 128
L = 16
NC = 2
NS = 16
NW = NC * NS
BPW = B // NW
CHUNK = 64
ROW_UNROLL = 8
PCOLS = 128
PROWS = B * L // PCOLS
PROWS_PW = BPW * L // PCOLS


def _sc_gather_dot(emb, ctx, src_idx, dst_idx):
    mesh = plsc.VectorSubcoreMesh(core_axis_name="c", subcore_axis_name="s")
    out_t = jax.ShapeDtypeStruct((PROWS, PCOLS), jnp.float32)

    @functools.partial(
        pl.kernel,
        out_type=out_t,
        mesh=mesh,
        scratch_types=[
            pltpu.VMEM((BPW,), jnp.int32),
            pltpu.VMEM((BPW,), jnp.int32),
            pltpu.VMEM((CHUNK, D), jnp.float32),
            pltpu.VMEM((CHUNK, D), jnp.float32),
            pltpu.VMEM((CHUNK, D), jnp.float32),
            pltpu.VMEM((CHUNK, D), jnp.float32),
            pltpu.VMEM((CHUNK, D), jnp.float32),
            pltpu.VMEM((CHUNK, D), jnp.float32),
            pltpu.VMEM((PROWS_PW, PCOLS), jnp.float32),
            pltpu.SemaphoreType.DMA,
            pltpu.SemaphoreType.DMA,
            pltpu.SemaphoreType.DMA,
            pltpu.SemaphoreType.DMA,
            pltpu.SemaphoreType.DMA,
            pltpu.SemaphoreType.DMA,
            pltpu.SemaphoreType.DMA,
            pltpu.SemaphoreType.DMA,
        ],
    )
    def k(emb_hbm, ctx_hbm, src_hbm, dst_hbm, out_hbm,
          sidx_v, didx_v, sbuf0, sbuf1, sbuf2, dbuf0, dbuf1, dbuf2, part_v,
          sem_s0, sem_s1, sem_s2, sem_d0, sem_d1, sem_d2, sem_i0, sem_i1):
        wid = lax.axis_index("s") * NC + lax.axis_index("c")
        base = wid * BPW
        ci0 = pltpu.async_copy(src_hbm.at[pl.ds(base, BPW)], sidx_v, sem_i0)
        ci1 = pltpu.async_copy(dst_hbm.at[pl.ds(base, BPW)], didx_v, sem_i1)
        ci0.wait()
        ci1.wait()

        sbufs = (sbuf0, sbuf1, sbuf2)
        dbufs = (dbuf0, dbuf1, dbuf2)
        ssems = (sem_s0, sem_s1, sem_s2)
        dsems = (sem_d0, sem_d1, sem_d2)
        DEPTH = 2
        CHUNKS = (64, 64, 64, 64, 64, 64, 64, 64)
        OFFS = tuple(sum(CHUNKS[:i]) for i in range(len(CHUNKS)))
        NCH = len(CHUNKS)

        def start(c):
            off = OFFS[c]
            n = CHUNKS[c]
            cs = pltpu.async_copy(
                emb_hbm.at[sidx_v.at[pl.ds(off, n)]],
                sbufs[c % DEPTH].at[pl.ds(0, n)],
                ssems[c % DEPTH])
            cd = pltpu.async_copy(
                ctx_hbm.at[didx_v.at[pl.ds(off, n)]],
                dbufs[c % DEPTH].at[pl.ds(0, n)],
                dsems[c % DEPTH])
            return cs, cd

        pend = [start(c) for c in range(DEPTH)]
        wouts = []
        for c in range(NCH):
            cs, cd = pend[c % DEPTH]
            cs.wait()
            cd.wait()
            sb = sbufs[c % DEPTH]
            db = dbufs[c % DEPTH]
            pbase = OFFS[c]

            @plsc.parallel_loop(0, CHUNKS[c], step=ROW_UNROLL, unroll=2)
            def _(r0):
                pline = (pbase + r0) // 8
                for p in range(ROW_UNROLL // 2):
                    ra = r0 + 2 * p
                    rb = ra + 1
                    acca = sb[ra, pl.ds(0, L)] * db[ra, pl.ds(0, L)]
                    accb = sb[rb, pl.ds(0, L)] * db[rb, pl.ds(0, L)]
                    for g in range(1, D // L):
                        acca += sb[ra, pl.ds(g * L, L)] * db[ra, pl.ds(g * L, L)]
                        accb += sb[rb, pl.ds(g * L, L)] * db[rb, pl.ds(g * L, L)]
                    ua, ub = 2 * p, 2 * p + 1
                    part_v[pline + ua // 8, pl.ds((ua % 8) * L, L)] = acca
                    part_v[pline + ub // 8, pl.ds((ub % 8) * L, L)] = accb

            if c + DEPTH < NCH:
                pend[c % DEPTH] = start(c + DEPTH)

            lo = OFFS[c] // 8
            n = CHUNKS[c] // 8
            wouts.append(pltpu.async_copy(
                part_v.at[pl.ds(lo, n)],
                out_hbm.at[pl.ds(wid * PROWS_PW + lo, n)],
                sem_i0))

        for w in wouts:
            w.wait()

    return k(emb, ctx, src_idx, dst_idx)


def _tc_loss_body(p_ref, o_ref):
    y = p_ref[...]
    r_grp = jax.lax.broadcasted_iota(jnp.int32, (PCOLS, PCOLS), 0) // L
    c_grp = jax.lax.broadcasted_iota(jnp.int32, (PCOLS, PCOLS), 1) // L
    g = (r_grp == c_grp).astype(jnp.float32)
    dot = jax.lax.dot_general(y, g, (((1,), (0,)), ((), ())),
                              preferred_element_type=jnp.float32)
    sp = jax.nn.softplus(-dot)
    o_ref[...] = (jnp.sum(sp) * (1.0 / (B * L))).reshape(1, 1)


def _tc_loss(p):
    out = pl.pallas_call(
        _tc_loss_body,
        out_shape=jax.ShapeDtypeStruct((1, 1), jnp.float32),
    )(p)
    return out[0, 0]


def kernel(src_nodes, dst_nodes, embedding, context_embedding):
    part = _sc_gather_dot(
        embedding,
        context_embedding,
        src_nodes.astype(jnp.int32),
        dst_nodes.astype(jnp.int32),
    )
    return _tc_loss(part)


# speedup vs baseline: 1.1844x; 1.0427x over previous
import functools

import jax
import jax.numpy as jnp
from jax import lax
from jax.experimental import pallas as pl
from jax.experimental.pallas import tpu as pltpu
from jax.experimental.pallas import tpu_sc as plsc

B = 16384
D = 128
L = 16
NC = 2
NS = 16
NW = NC * NS
BPW = B // NW
CHUNK = 64
ROW_UNROLL = 8
PCOLS = 128
PROWS = B * L // PCOLS
PROWS_PW = BPW * L // PCOLS


def _sc_gather_dot(emb, ctx, src_idx, dst_idx):
    mesh = plsc.VectorSubcoreMesh(core_axis_name="c", subcore_axis_name="s")
    out_t = jax.ShapeDtypeStruct((PROWS, PCOLS), jnp.float32)

    @functools.partial(
        pl.kernel,
        out_type=out_t,
        mesh=mesh,
        scratch_types=[
            pltpu.VMEM((BPW,), jnp.int32),
            pltpu.VMEM((BPW,), jnp.int32),
            pltpu.VMEM((CHUNK, D), jnp.float32),
            pltpu.VMEM((CHUNK, D), jnp.float32),
            pltpu.VMEM((CHUNK, D), jnp.float32),
            pltpu.VMEM((CHUNK, D), jnp.float32),
            pltpu.VMEM((CHUNK, D), jnp.float32),
            pltpu.VMEM((CHUNK, D), jnp.float32),
            pltpu.VMEM((PROWS_PW, PCOLS), jnp.float32),
            pltpu.SemaphoreType.DMA,
            pltpu.SemaphoreType.DMA,
            pltpu.SemaphoreType.DMA,
            pltpu.SemaphoreType.DMA,
            pltpu.SemaphoreType.DMA,
            pltpu.SemaphoreType.DMA,
            pltpu.SemaphoreType.DMA,
            pltpu.SemaphoreType.DMA,
        ],
    )
    def k(emb_hbm, ctx_hbm, src_hbm, dst_hbm, out_hbm,
          sidx_v, didx_v, sbuf0, sbuf1, sbuf2, dbuf0, dbuf1, dbuf2, part_v,
          sem_s0, sem_s1, sem_s2, sem_d0, sem_d1, sem_d2, sem_i0, sem_i1):
        wid = lax.axis_index("s") * NC + lax.axis_index("c")
        base = wid * BPW
        ci0 = pltpu.async_copy(src_hbm.at[pl.ds(base, BPW)], sidx_v, sem_i0)
        ci1 = pltpu.async_copy(dst_hbm.at[pl.ds(base, BPW)], didx_v, sem_i1)
        ci0.wait()
        ci1.wait()

        sbufs = (sbuf0, sbuf1, sbuf2)
        dbufs = (dbuf0, dbuf1, dbuf2)
        ssems = (sem_s0, sem_s1, sem_s2)
        dsems = (sem_d0, sem_d1, sem_d2)
        DEPTH = 3
        CHUNKS = (64, 64, 64, 64, 64, 64, 64, 64)
        OFFS = tuple(sum(CHUNKS[:i]) for i in range(len(CHUNKS)))
        NCH = len(CHUNKS)

        def start(c):
            off = OFFS[c]
            n = CHUNKS[c]
            cs = pltpu.async_copy(
                emb_hbm.at[sidx_v.at[pl.ds(off, n)]],
                sbufs[c % DEPTH].at[pl.ds(0, n)],
                ssems[c % DEPTH])
            cd = pltpu.async_copy(
                ctx_hbm.at[didx_v.at[pl.ds(off, n)]],
                dbufs[c % DEPTH].at[pl.ds(0, n)],
                dsems[c % DEPTH])
            return cs, cd

        pend = [start(c) for c in range(DEPTH)]
        wouts = []
        for c in range(NCH):
            cs, cd = pend[c % DEPTH]
            cs.wait()
            cd.wait()
            sb = sbufs[c % DEPTH]
            db = dbufs[c % DEPTH]
            pbase = OFFS[c]

            @plsc.parallel_loop(0, CHUNKS[c], step=ROW_UNROLL, unroll=2)
            def _(r0):
                pline = (pbase + r0) // 8
                for p in range(ROW_UNROLL // 2):
                    ra = r0 + 2 * p
                    rb = ra + 1
                    acca = sb[ra, pl.ds(0, L)] * db[ra, pl.ds(0, L)]
                    accb = sb[rb, pl.ds(0, L)] * db[rb, pl.ds(0, L)]
                    for g in range(1, D // L):
                        acca += sb[ra, pl.ds(g * L, L)] * db[ra, pl.ds(g * L, L)]
                        accb += sb[rb, pl.ds(g * L, L)] * db[rb, pl.ds(g * L, L)]
                    ua, ub = 2 * p, 2 * p + 1
                    part_v[pline + ua // 8, pl.ds((ua % 8) * L, L)] = acca
                    part_v[pline + ub // 8, pl.ds((ub % 8) * L, L)] = accb

            if c + DEPTH < NCH:
                pend[c % DEPTH] = start(c + DEPTH)

            lo = OFFS[c] // 8
            n = CHUNKS[c] // 8
            wouts.append(pltpu.async_copy(
                part_v.at[pl.ds(lo, n)],
                out_hbm.at[pl.ds(wid * PROWS_PW + lo, n)],
                sem_i0))

        for w in wouts:
            w.wait()

    return k(emb, ctx, src_idx, dst_idx)


def _tc_loss_body(p_ref, o_ref):
    y = p_ref[...]
    r_grp = jax.lax.broadcasted_iota(jnp.int32, (PCOLS, PCOLS), 0) // L
    c_grp = jax.lax.broadcasted_iota(jnp.int32, (PCOLS, PCOLS), 1) // L
    g = (r_grp == c_grp).astype(jnp.float32)
    dot = jax.lax.dot_general(y, g, (((1,), (0,)), ((), ())),
                              preferred_element_type=jnp.float32)
    sp = jax.nn.softplus(-dot)
    o_ref[...] = (jnp.sum(sp) * (1.0 / (B * L))).reshape(1, 1)


def _tc_loss(p):
    out = pl.pallas_call(
        _tc_loss_body,
        out_shape=jax.ShapeDtypeStruct((1, 1), jnp.float32),
    )(p)
    return out[0, 0]


def kernel(src_nodes, dst_nodes, embedding, context_embedding):
    part = _sc_gather_dot(
        embedding,
        context_embedding,
        src_nodes.astype(jnp.int32),
        dst_nodes.astype(jnp.int32),
    )
    return _tc_loss(part)
